# Initial kernel scaffold; baseline (speedup 1.0000x reference)
import functools

import jax
import jax.numpy as jnp
from jax import lax
from jax.experimental import pallas as pl
from jax.experimental.pallas import tpu as pltpu
from jax.experimental.pallas import tpu_sc as plsc

T, N, E, F = 4, 10000, 160000, 256
H = F // 2
NC, NS = 2, 16
TN = T * N

TN_PAD = 40960
DW = NC * NS
DPER = T * E // DW
DCH = 80
DNC = DPER // DCH
DSL = TN_PAD // NS

EPT = E // NS
CH = 80
NCH = EPT // CH
RPT = N // NS

_mesh = plsc.VectorSubcoreMesh(core_axis_name="c", subcore_axis_name="s")


def _deg_body(cols_hbm, ones_hbm, zeros_hbm, deg_out, idx_v, ones_v, deg_sp):
    c = lax.axis_index("c")
    s = lax.axis_index("s")
    w = c * NS + s
    pltpu.sync_copy(zeros_hbm, deg_sp.at[pl.ds(s * DSL, DSL)])
    pltpu.sync_copy(ones_hbm, ones_v)
    pltpu.sync_copy(cols_hbm.at[w], idx_v)
    plsc.subcore_barrier()

    def chunk(j, carry):
        pltpu.sync_copy(ones_v, deg_sp.at[idx_v.at[j]], add=True)
        return carry

    lax.fori_loop(0, DNC, chunk, 0)
    plsc.subcore_barrier()
    pltpu.sync_copy(deg_sp.at[pl.ds(s * DSL, DSL)],
                    deg_out.at[c, pl.ds(s * DSL, DSL)])


_deg_kernel = functools.partial(
    pl.kernel,
    out_type=jax.ShapeDtypeStruct((NC, TN_PAD, 1), jnp.float32),
    mesh=_mesh,
    scratch_types=[
        pltpu.VMEM((DNC, DCH), jnp.int32),
        pltpu.VMEM((DCH, 1), jnp.float32),
        pltpu.VMEM_SHARED((TN_PAD, 1), jnp.float32),
    ],
)(_deg_body)


def _mm_body(x_ref, w_ref, degp_ref, s2_ref, dinv_ref):
    xw = jnp.dot(x_ref[...], w_ref[...], preferred_element_type=jnp.float32)
    deg = degp_ref[0] + degp_ref[1] + 1.0
    dinv = lax.rsqrt(deg)
    sc = xw * dinv
    s2_ref[0] = sc[:, :H]
    s2_ref[1] = sc[:, H:]
    dinv_ref[...] = dinv


def _run_mm(x2d, w, degp):
    R = 2000
    grid = (TN // R,)
    return pl.pallas_call(
        _mm_body,
        grid=grid,
        in_specs=[
            pl.BlockSpec((R, F), lambda g: (g, 0)),
            pl.BlockSpec((F, F), lambda g: (0, 0)),
            pl.BlockSpec((NC, R, 1), lambda g: (0, g, 0)),
        ],
        out_specs=[
            pl.BlockSpec((NC, R, H), lambda g: (0, g, 0)),
            pl.BlockSpec((R, 1), lambda g: (g, 0)),
        ],
        out_shape=[
            jax.ShapeDtypeStruct((NC, TN, H), jnp.float32),
            jax.ShapeDtypeStruct((TN, 1), jnp.float32),
        ],
    )(x2d, w, degp)


def _gat_body(rows_hbm, cols_hbm, s2f_hbm, out_hbm, rows_v, cols_v, gbuf, acc_sp):
    c = lax.axis_index("c")
    s = lax.axis_index("s")
    for t in range(T):
        pltpu.sync_copy(s2f_hbm.at[pl.ds(c * TN + t * N + s * RPT, RPT)],
                        acc_sp.at[pl.ds(s * RPT, RPT)])
        pltpu.sync_copy(rows_hbm.at[c, t, s], rows_v)
        pltpu.sync_copy(cols_hbm.at[t, s], cols_v)
        plsc.subcore_barrier()

        def chunk(j, carry):
            pltpu.sync_copy(s2f_hbm.at[rows_v.at[j]], gbuf)
            pltpu.sync_copy(gbuf, acc_sp.at[cols_v.at[j]], add=True)
            return carry

        lax.fori_loop(0, NCH, chunk, 0)
        plsc.subcore_barrier()
        pltpu.sync_copy(acc_sp.at[pl.ds(s * RPT, RPT)],
                        out_hbm.at[t, pl.ds(s * RPT, RPT), pl.ds(c * H, H)])


_gat_kernel = functools.partial(
    pl.kernel,
    out_type=jax.ShapeDtypeStruct((T, N, F), jnp.float32),
    mesh=_mesh,
    scratch_types=[
        pltpu.VMEM((NCH, CH), jnp.int32),
        pltpu.VMEM((NCH, CH), jnp.int32),
        pltpu.VMEM((CH, H), jnp.float32),
        pltpu.VMEM_SHARED((N, H), jnp.float32),
    ],
)(_gat_body)


def _fin_body(acc_ref, dinv_ref, b_ref, o_ref):
    o_ref[...] = acc_ref[...] * dinv_ref[...] + b_ref[...]


def _run_fin(acc2d, dinv, b2d):
    R = 2000
    grid = (TN // R,)
    return pl.pallas_call(
        _fin_body,
        grid=grid,
        in_specs=[
            pl.BlockSpec((R, F), lambda g: (g, 0)),
            pl.BlockSpec((R, 1), lambda g: (g, 0)),
            pl.BlockSpec((1, F), lambda g: (0, 0)),
        ],
        out_specs=pl.BlockSpec((R, F), lambda g: (g, 0)),
        out_shape=jax.ShapeDtypeStruct((TN, F), jnp.float32),
    )(acc2d, dinv, b2d)


def kernel(A_list, node_feats_list, W_gcn, b_gcn, gcn_weights,
           w_ih0, w_hh0, b_ih0, b_hh0, w_ih1, w_hh1, b_ih1, b_hh1):
    rows = A_list[:, 0, :]
    cols = A_list[:, 1, :]
    toff = (jnp.arange(T, dtype=jnp.int32) * N)[:, None]

    cols_deg = (cols + toff).reshape(DW, DNC, DCH)
    ones = jnp.ones((DCH, 1), jnp.float32)
    zeros = jnp.zeros((DSL, 1), jnp.float32)
    degp = _deg_kernel(cols_deg, ones, zeros)

    x2d = node_feats_list.reshape(TN, F)
    s2, dinv = _run_mm(x2d, W_gcn, degp[:, :TN, :])
    s2f = s2.reshape(NC * TN, H)

    rows_t = (rows + toff).reshape(1, T, NS, NCH, CH)
    coff = (jnp.arange(NC, dtype=jnp.int32) * TN).reshape(NC, 1, 1, 1, 1)
    rows2 = rows_t + coff
    cols3 = cols.reshape(T, NS, NCH, CH)

    acc = _gat_kernel(rows2, cols3, s2f)

    out2d = _run_fin(acc.reshape(TN, F), dinv, b_gcn.reshape(1, F))
    return out2d.reshape(T, N, F)


# speedup vs baseline: 11.3957x; 11.3957x over previous
import functools

import jax
import jax.numpy as jnp
from jax import lax
from jax.experimental import pallas as pl
from jax.experimental.pallas import tpu as pltpu
from jax.experimental.pallas import tpu_sc as plsc

T, N, E, F = 4, 10000, 160000, 256
H = F // 2
NC, NS = 2, 16
TN = T * N

DW = NC * NS
DEPT = E // DW
CHD = 100
NCD = DEPT // CHD
DCW = 16

EPT = E // NS
CH = 80
NCH = EPT // CH
SEG = 5
SCH = NCH // SEG
RPT = 624
REM_OFF = NS * RPT
REM = N - REM_OFF

_mesh = plsc.VectorSubcoreMesh(core_axis_name="c", subcore_axis_name="s")


DBB = 104
DNBB = 624 // DBB


def _deg_body(cols_hbm, ones_hbm, zeros_hbm, deg_out, cols_v, ones_v, zb_v,
              acc_sp):
    c = lax.axis_index("c")
    s = lax.axis_index("s")
    pltpu.sync_copy(ones_hbm, ones_v)
    pltpu.sync_copy(zeros_hbm, zb_v)
    for t in range(T):
        for k in range(DNBB):
            pltpu.sync_copy(zb_v, acc_sp.at[pl.ds(s * RPT + k * DBB, DBB)])

        @pl.when(s == NS - 1)
        def _():
            pltpu.sync_copy(zb_v.at[pl.ds(0, REM)], acc_sp.at[pl.ds(REM_OFF, REM)])

        pltpu.sync_copy(cols_hbm.at[t, c, s], cols_v)
        plsc.subcore_barrier()

        def chunk(j, carry):
            pltpu.sync_copy(ones_v, acc_sp.at[cols_v.at[j]], add=True)
            return carry

        lax.fori_loop(0, NCD, chunk, 0)
        plsc.subcore_barrier()
        for k in range(DNBB):
            pltpu.sync_copy(acc_sp.at[pl.ds(s * RPT + k * DBB, DBB)], zb_v)
            pltpu.sync_copy(zb_v, deg_out.at[c, t, pl.ds(s * RPT + k * DBB, DBB)])

        @pl.when(s == NS - 1)
        def _():
            pltpu.sync_copy(acc_sp.at[pl.ds(REM_OFF, REM)], zb_v.at[pl.ds(0, REM)])
            pltpu.sync_copy(zb_v.at[pl.ds(0, REM)],
                            deg_out.at[c, t, pl.ds(REM_OFF, REM)])

        pltpu.sync_copy(zeros_hbm, zb_v)


_deg_kernel = functools.partial(
    pl.kernel,
    out_type=jax.ShapeDtypeStruct((NC, T, N, H), jnp.float32),
    mesh=_mesh,
    scratch_types=[
        pltpu.VMEM((NCD, CHD), jnp.int32),
        pltpu.VMEM((CHD, H), jnp.float32),
        pltpu.VMEM((DBB, H), jnp.float32),
        pltpu.VMEM_SHARED((N, H), jnp.float32),
    ],
)(_deg_body)


def _mm_body(x_ref, w_ref, degp_ref, s2_ref, dinv_ref):
    xw = jnp.dot(x_ref[...], w_ref[...], preferred_element_type=jnp.float32)
    deg = degp_ref[0] + degp_ref[1] + 1.0
    dinv = lax.rsqrt(deg)
    sc = xw * dinv
    s2_ref[0] = sc[:, :H]
    s2_ref[1] = sc[:, H:]
    dinv_ref[...] = dinv


def _run_mm(x2d, w, degp):
    R = 2000
    grid = (TN // R,)
    return pl.pallas_call(
        _mm_body,
        grid=grid,
        in_specs=[
            pl.BlockSpec((R, F), lambda g: (g, 0)),
            pl.BlockSpec((F, F), lambda g: (0, 0)),
            pl.BlockSpec((NC, R, 1), lambda g: (0, g, 0)),
        ],
        out_specs=[
            pl.BlockSpec((NC, R, H), lambda g: (0, g, 0)),
            pl.BlockSpec((R, 1), lambda g: (g, 0)),
        ],
        out_shape=[
            jax.ShapeDtypeStruct((NC, TN, H), jnp.float32),
            jax.ShapeDtypeStruct((TN, 1), jnp.float32),
        ],
    )(x2d, w, degp)


BB = 104
NBB = RPT // BB


def _gat_body(rows_hbm, cols_hbm, s2f_hbm, out_hbm, rows_v, cols_v, gbuf, bb_v,
              acc_sp):
    c = lax.axis_index("c")
    s = lax.axis_index("s")
    for t in range(T):
        for k in range(NBB):
            pltpu.sync_copy(
                s2f_hbm.at[pl.ds(c * TN + t * N + s * RPT + k * BB, BB)], bb_v)
            pltpu.sync_copy(bb_v, acc_sp.at[pl.ds(s * RPT + k * BB, BB)])

        @pl.when(s == NS - 1)
        def _():
            pltpu.sync_copy(s2f_hbm.at[pl.ds(c * TN + t * N + REM_OFF, REM)],
                            bb_v.at[pl.ds(0, REM)])
            pltpu.sync_copy(bb_v.at[pl.ds(0, REM)], acc_sp.at[pl.ds(REM_OFF, REM)])

        plsc.subcore_barrier()

        def chunk(j, carry):
            pltpu.sync_copy(s2f_hbm.at[rows_v.at[j]], gbuf)
            pltpu.sync_copy(gbuf, acc_sp.at[cols_v.at[j]], add=True)
            return carry

        for seg in range(SEG):
            pltpu.sync_copy(rows_hbm.at[c, t, s, seg], rows_v)
            pltpu.sync_copy(cols_hbm.at[t, s, seg], cols_v)
            lax.fori_loop(0, SCH, chunk, 0)
        plsc.subcore_barrier()
        for k in range(NBB):
            pltpu.sync_copy(acc_sp.at[pl.ds(s * RPT + k * BB, BB)], bb_v)
            pltpu.sync_copy(
                bb_v, out_hbm.at[t, pl.ds(s * RPT + k * BB, BB), pl.ds(c * H, H)])

        @pl.when(s == NS - 1)
        def _():
            pltpu.sync_copy(acc_sp.at[pl.ds(REM_OFF, REM)], bb_v.at[pl.ds(0, REM)])
            pltpu.sync_copy(bb_v.at[pl.ds(0, REM)],
                            out_hbm.at[t, pl.ds(REM_OFF, REM), pl.ds(c * H, H)])


_gat_kernel = functools.partial(
    pl.kernel,
    out_type=jax.ShapeDtypeStruct((T, N, F), jnp.float32),
    mesh=_mesh,
    scratch_types=[
        pltpu.VMEM((SCH, CH), jnp.int32),
        pltpu.VMEM((SCH, CH), jnp.int32),
        pltpu.VMEM((CH, H), jnp.float32),
        pltpu.VMEM((BB, H), jnp.float32),
        pltpu.VMEM_SHARED((N, H), jnp.float32),
    ],
)(_gat_body)


def _fin_body(acc_ref, dinv_ref, b_ref, o_ref):
    o_ref[...] = acc_ref[...] * dinv_ref[...] + b_ref[...]


def _run_fin(acc2d, dinv, b2d):
    R = 2000
    grid = (TN // R,)
    return pl.pallas_call(
        _fin_body,
        grid=grid,
        in_specs=[
            pl.BlockSpec((R, F), lambda g: (g, 0)),
            pl.BlockSpec((R, 1), lambda g: (g, 0)),
            pl.BlockSpec((1, F), lambda g: (0, 0)),
        ],
        out_specs=pl.BlockSpec((R, F), lambda g: (g, 0)),
        out_shape=jax.ShapeDtypeStruct((TN, F), jnp.float32),
    )(acc2d, dinv, b2d)


def kernel(A_list, node_feats_list, W_gcn, b_gcn, gcn_weights,
           w_ih0, w_hh0, b_ih0, b_hh0, w_ih1, w_hh1, b_ih1, b_hh1):
    rows = A_list[:, 0, :]
    cols = A_list[:, 1, :]
    toff = (jnp.arange(T, dtype=jnp.int32) * N)[:, None]

    cols_deg = cols.reshape(T, NC, NS, NCD, CHD)
    ones_d = jnp.ones((CHD, H), jnp.float32)
    zeros_d = jnp.zeros((DBB, H), jnp.float32)
    deg4 = _deg_kernel(cols_deg, ones_d, zeros_d)
    degp = deg4.reshape(NC, TN, H)[:, :, :1]

    x2d = node_feats_list.reshape(TN, F)
    s2, dinv = _run_mm(x2d, W_gcn, degp)
    s2f = s2.reshape(NC * TN, H)

    rows_t = (rows + toff).reshape(1, T, NS, SEG, SCH, CH)
    coff = (jnp.arange(NC, dtype=jnp.int32) * TN).reshape(NC, 1, 1, 1, 1, 1)
    rows2 = rows_t + coff
    cols3 = cols.reshape(T, NS, SEG, SCH, CH)

    acc = _gat_kernel(rows2, cols3, s2f)

    out2d = _run_fin(acc.reshape(TN, F), dinv, b_gcn.reshape(1, F))
    return out2d.reshape(T, N, F)
